# baseline (speedup 1.0000x reference)
import functools

import jax
import jax.numpy as jnp
from jax import lax
from jax.experimental import pallas as pl
from jax.experimental.pallas import tpu as pltpu
from jax.experimental.pallas import tpu_sc as plsc

NN = 10000
NE = 320000
HID = 16
HEADS = 8
FFD = HID * HEADS
NG = 64
NCLS = 10
EPS = 1e-5

NC, NS = 2, 16
NW = NC * NS
NPAD = 10240
RT = NPAD // NW
CAP = 16384
CHUNK = 12800
EC = 128

_mesh = plsc.VectorSubcoreMesh(core_axis_name="c", subcore_axis_name="s")
_sc_params = pltpu.CompilerParams(needs_layout_passes=False)


def _wid():
    return lax.axis_index("s") * NC + lax.axis_index("c")


def _partition_body(ei_hbm, esrc_hbm, edst_hbm, ecnt_hbm,
                    ei_v, out_src, out_dst, cnt_v):
    wid = _wid()
    lo = wid * RT
    lane = lax.iota(jnp.int32, 16)

    def chunk_body(ci, cnt):
        pltpu.sync_copy(ei_hbm.at[:, pl.ds(ci * CHUNK, CHUNK)], ei_v)

        def grp(gi, cnt):
            for u in range(4):
                d16 = ei_v[1, pl.ds(gi * 64 + u * 16, 16)]
                s16 = ei_v[0, pl.ds(gi * 64 + u * 16, 16)]
                m = (d16 >= lo) & (d16 < lo + RT)
                pos = plsc.cumsum(m.astype(jnp.int32))
                idx = jnp.where(m, cnt + pos - 1, CAP + lane)
                plsc.store_scatter(out_src, [idx], s16)
                plsc.store_scatter(out_dst, [idx], d16 - lo)
                cnt = cnt + pos[15]
            return cnt

        return lax.fori_loop(0, CHUNK // 64, grp, cnt)

    cnt = lax.fori_loop(0, NE // CHUNK, chunk_body, jnp.int32(0))

    z16 = jnp.zeros((16,), jnp.int32)
    d16 = jnp.full((16,), RT, jnp.int32)

    def ztail(i, _):
        out_src[pl.ds(cnt + i * 16, 16)] = z16
        out_dst[pl.ds(cnt + i * 16, 16)] = d16
        return 0

    lax.fori_loop(0, EC // 16, ztail, 0)

    cnt_v[...] = jnp.full((16,), cnt, jnp.int32)
    pltpu.sync_copy(cnt_v, ecnt_hbm.at[wid])
    pltpu.sync_copy(out_src.at[pl.ds(0, CAP)], esrc_hbm.at[wid])
    pltpu.sync_copy(out_dst.at[pl.ds(0, CAP)], edst_hbm.at[wid])


_partition = functools.partial(
    pl.kernel,
    mesh=_mesh,
    compiler_params=_sc_params,
    out_type=[
        jax.ShapeDtypeStruct((NW, CAP), jnp.int32),
        jax.ShapeDtypeStruct((NW, CAP), jnp.int32),
        jax.ShapeDtypeStruct((NW, 16), jnp.int32),
    ],
    scratch_types=[
        pltpu.VMEM((2, CHUNK), jnp.int32),
        pltpu.VMEM((CAP + 16,), jnp.int32),
        pltpu.VMEM((CAP + 16,), jnp.int32),
        pltpu.VMEM((16,), jnp.int32),
    ],
)(_partition_body)


def _edge_body(hs_hbm, hd_hbm, esrc_hbm, edst_hbm, ecnt_hbm, att_hbm,
               num_hbm, den_hbm, hd_loc, acc_num, acc_den, rows0, rows1,
               srcv0, srcv1, dstv0, dstv1, wbuf0, wbuf1, attv, cntv,
               sem0, sem1):
    wid = _wid()
    base = wid * RT
    pltpu.sync_copy(hd_hbm.at[pl.ds(base, RT)], hd_loc.at[pl.ds(0, RT)])
    pltpu.sync_copy(att_hbm, attv)
    pltpu.sync_copy(ecnt_hbm.at[wid], cntv)
    cnt = cntv[...][0]

    zf = jnp.zeros((16,), jnp.float32)

    def zrow(r, _):
        for g in range(8):
            acc_num[r, pl.ds(g * 16, 16)] = zf
        acc_den[pl.ds(r * 16, 16)] = zf
        return 0

    lax.fori_loop(0, RT, zrow, 0)
    for g in range(8):
        hd_loc[RT, pl.ds(g * 16, 16)] = zf

    atts = [attv[pl.ds(h * 16, 16)] for h in range(HEADS)]
    lane = lax.iota(jnp.int32, 16)
    i15 = jnp.full((16,), 15, jnp.int32)
    ih = [jnp.full((16,), h, jnp.int32) for h in range(HEADS)]

    nch = (cnt + EC - 1) // EC

    def fetch(ci, srcv, dstv, rows, sem):
        off = ci * EC
        pltpu.sync_copy(esrc_hbm.at[wid, pl.ds(off, EC)], srcv)
        pltpu.sync_copy(edst_hbm.at[wid, pl.ds(off, EC)], dstv.at[pl.ds(0, EC)])
        pltpu.async_copy(hs_hbm.at[srcv], rows, sem)

    def compute(rows, dstv, wbuf):
        def grp1(g, _):
            dl16 = dstv[pl.ds(g * 8, 16)]
            for j in range(8):
                e = g * 8 + j
                dl = dl16[j]
                lv = zf
                for h in range(HEADS):
                    a = rows[e, pl.ds(h * 16, 16)]
                    b = hd_loc[dl, pl.ds(h * 16, 16)]
                    t = a + b
                    t = jnp.maximum(t, 0.2 * t)
                    c = plsc.cumsum(t * atts[h])
                    tot = c[i15]
                    lv = jnp.where(lane == h, tot, lv)
                wv = jnp.exp(lv)
                wbuf[pl.ds(e * 16, 16)] = wv
                plsc.addupdate(acc_den.at[pl.ds(dl * 16, 16)], wv)
            return 0

        def grp2(g, _):
            dl16 = dstv[pl.ds(g * 8, 16)]
            for j in range(8):
                e = g * 8 + j
                dl = dl16[j]
                wv = wbuf[pl.ds(e * 16, 16)]
                for h in range(HEADS):
                    a = rows[e, pl.ds(h * 16, 16)]
                    wh = wv[ih[h]]
                    plsc.addupdate(acc_num.at[dl, pl.ds(h * 16, 16)], a * wh)
            return 0

        lax.fori_loop(0, EC // 8, grp1, 0)
        lax.fori_loop(0, EC // 8, grp2, 0)

    @pl.when(nch > 0)
    def _():
        fetch(0, srcv0, dstv0, rows0, sem0)

    def pair(i, _):
        c0 = 2 * i

        @pl.when(c0 + 1 < nch)
        def _():
            fetch(c0 + 1, srcv1, dstv1, rows1, sem1)

        pltpu.make_async_copy(hs_hbm.at[srcv0], rows0, sem0).wait()
        compute(rows0, dstv0, wbuf0)

        @pl.when(c0 + 2 < nch)
        def _():
            fetch(c0 + 2, srcv0, dstv0, rows0, sem0)

        @pl.when(c0 + 1 < nch)
        def _():
            pltpu.make_async_copy(hs_hbm.at[srcv1], rows1, sem1).wait()
            compute(rows1, dstv1, wbuf1)

        return 0

    lax.fori_loop(0, (nch + 1) // 2, pair, 0)
    pltpu.sync_copy(acc_num.at[pl.ds(0, RT)], num_hbm.at[pl.ds(base, RT)])
    pltpu.sync_copy(acc_den.at[pl.ds(0, RT * 16)], den_hbm.at[wid])


_edge = functools.partial(
    pl.kernel,
    mesh=_mesh,
    compiler_params=_sc_params,
    out_type=[
        jax.ShapeDtypeStruct((NPAD, 128), jnp.float32),
        jax.ShapeDtypeStruct((NW, RT * 16), jnp.float32),
    ],
    scratch_types=[
        pltpu.VMEM((RT + 8, 128), jnp.float32),
        pltpu.VMEM((RT + 8, 128), jnp.float32),
        pltpu.VMEM(((RT + 8) * 16,), jnp.float32),
        pltpu.VMEM((EC, 128), jnp.float32),
        pltpu.VMEM((EC, 128), jnp.float32),
        pltpu.VMEM((EC,), jnp.int32),
        pltpu.VMEM((EC,), jnp.int32),
        pltpu.VMEM((EC + 16,), jnp.int32),
        pltpu.VMEM((EC + 16,), jnp.int32),
        pltpu.VMEM((EC * 16,), jnp.float32),
        pltpu.VMEM((EC * 16,), jnp.float32),
        pltpu.VMEM((128,), jnp.float32),
        pltpu.VMEM((16,), jnp.int32),
        pltpu.SemaphoreType.DMA,
        pltpu.SemaphoreType.DMA,
    ],
)(_edge_body)


def _pool_body(h_hbm, batch_hbm, psum_hbm, pmax_hbm, hl, bl, asum, amax):
    wid = _wid()
    base = wid * RT
    pltpu.sync_copy(h_hbm.at[pl.ds(base, RT)], hl)
    pltpu.sync_copy(batch_hbm.at[pl.ds(base, RT)], bl.at[pl.ds(0, RT)])

    zf = jnp.zeros((16,), jnp.float32)
    ninf = jnp.full((16,), -jnp.inf, jnp.float32)

    def zrow(g, _):
        for j in range(9):
            asum[g, pl.ds(j * 16, 16)] = zf
        for j in range(8):
            amax[g, pl.ds(j * 16, 16)] = ninf
        return 0

    lax.fori_loop(0, NG, zrow, 0)

    lane = lax.iota(jnp.int32, 16)
    one0 = jnp.where(lane == 0, 1.0, 0.0).astype(jnp.float32)
    nrows = jnp.minimum(RT, NN - base)

    def row(r, _):
        g = bl[pl.ds(r, 16)][0]
        for j in range(8):
            v = hl[r, pl.ds(j * 16, 16)]
            plsc.addupdate(asum.at[g, pl.ds(j * 16, 16)], v)
            amax[g, pl.ds(j * 16, 16)] = jnp.maximum(amax[g, pl.ds(j * 16, 16)], v)
        plsc.addupdate(asum.at[g, pl.ds(128, 16)], one0)
        return 0

    lax.fori_loop(0, nrows, row, 0)

    pltpu.sync_copy(asum, psum_hbm.at[wid])
    pltpu.sync_copy(amax, pmax_hbm.at[wid])


_pool = functools.partial(
    pl.kernel,
    mesh=_mesh,
    compiler_params=_sc_params,
    out_type=[
        jax.ShapeDtypeStruct((NW, NG, 144), jnp.float32),
        jax.ShapeDtypeStruct((NW, NG, 128), jnp.float32),
    ],
    scratch_types=[
        pltpu.VMEM((RT, 128), jnp.float32),
        pltpu.VMEM((RT + 16,), jnp.int32),
        pltpu.VMEM((NG, 144), jnp.float32),
        pltpu.VMEM((NG, 128), jnp.float32),
    ],
)(_pool_body)


_PREC = lax.Precision.HIGHEST


def _dot(a, b):
    return jnp.dot(a, b, precision=_PREC, preferred_element_type=jnp.float32)


def _ln(z, g, b):
    mu = jnp.mean(z, axis=-1, keepdims=True)
    var = jnp.mean((z - mu) ** 2, axis=-1, keepdims=True)
    return (z - mu) / jnp.sqrt(var + EPS) * g + b


def _gelu(z):
    return 0.5 * z * (1.0 + lax.erf(z * 0.7071067811865476))


BR = 1024


def _pre_body(x_ref, pw_ref, pb_ref, ws_ref, wd_ref, id_ref, hs_ref, hd_ref):
    x = x_ref[...]
    id_ref[...] = _dot(x, pw_ref[...]) + pb_ref[...]
    hs_ref[...] = _dot(x, ws_ref[...])
    hd_ref[...] = _dot(x, wd_ref[...])


def _tc_pre(xp, pw, pb, ws, wd):
    grid = (NPAD // BR,)
    blk = lambda r, c: pl.BlockSpec((r, c), lambda i: (0, 0))
    row = lambda c: pl.BlockSpec((BR, c), lambda i: (i, 0))
    return pl.pallas_call(
        _pre_body,
        grid=grid,
        in_specs=[row(128), blk(128, 128), blk(1, 128), blk(128, 128), blk(128, 128)],
        out_specs=[row(128), row(128), row(128)],
        out_shape=[jax.ShapeDtypeStruct((NPAD, 128), jnp.float32)] * 3,
    )(xp, pw, pb, ws, wd)


def _mid_body(has_next, num_ref, den_ref, hid_ref, cb_ref, bng_ref, bnb_ref, ls_ref,
              w1_ref, b1_ref, lng_ref, lnb_ref, w2_ref, b2_ref, *rest):
    if has_next:
        wsn_ref, wdn_ref, h_ref, hs_ref, hd_ref = rest
    else:
        (h_ref,) = rest
    num = num_ref[...]
    den16 = den_ref[...]
    sel = (lax.broadcasted_iota(jnp.int32, (16, 128), 1) // 16
           == lax.broadcasted_iota(jnp.int32, (16, 128), 0)).astype(jnp.float32)
    den = _dot(den16, sel)
    gat = num / (den + 1e-16) + cb_ref[...]
    a = _gelu(gat / jnp.sqrt(1.0 + EPS) * bng_ref[...] + bnb_ref[...])
    ls = ls_ref[0, 0]
    h = hid_ref[...] + ls * a
    z = _dot(h, w1_ref[...]) + b1_ref[...]
    z = _ln(z, lng_ref[...], lnb_ref[...])
    z = _gelu(z)
    ff = _dot(z, w2_ref[...]) + b2_ref[...]
    h2 = h + ls * ff
    h_ref[...] = h2
    if has_next:
        hs_ref[...] = _dot(h2, wsn_ref[...])
        hd_ref[...] = _dot(h2, wdn_ref[...])


def _tc_mid(num, den, hid, cb, bng, bnb, ls, w1, b1, lng, lnb, w2, b2, wsn=None, wdn=None):
    has_next = wsn is not None
    grid = (NPAD // BR,)
    blk = lambda r, c: pl.BlockSpec((r, c), lambda i: (0, 0))
    row = lambda c: pl.BlockSpec((BR, c), lambda i: (i, 0))
    in_specs = [row(128), row(16), row(128), blk(1, 128), blk(1, 128), blk(1, 128),
                blk(1, 1), blk(128, 256), blk(1, 256), blk(1, 256), blk(1, 256),
                blk(256, 128), blk(1, 128)]
    args = [num, den, hid, cb, bng, bnb, ls, w1, b1, lng, lnb, w2, b2]
    if has_next:
        in_specs += [blk(128, 128), blk(128, 128)]
        args += [wsn, wdn]
        out_specs = [row(128)] * 3
        out_shape = [jax.ShapeDtypeStruct((NPAD, 128), jnp.float32)] * 3
    else:
        out_specs = [row(128)]
        out_shape = [jax.ShapeDtypeStruct((NPAD, 128), jnp.float32)]
    return pl.pallas_call(
        functools.partial(_mid_body, has_next),
        grid=grid,
        in_specs=in_specs,
        out_specs=out_specs,
        out_shape=out_shape,
    )(*args)


def _post_body(num_ref, den_ref, hid_ref, cb_ref, bng_ref, bnb_ref, ls_ref, h_ref):
    num = num_ref[...]
    den16 = den_ref[...]
    sel = (lax.broadcasted_iota(jnp.int32, (16, 128), 1) // 16
           == lax.broadcasted_iota(jnp.int32, (16, 128), 0)).astype(jnp.float32)
    den = _dot(den16, sel)
    gat = num / (den + 1e-16) + cb_ref[...]
    a = _gelu(gat / jnp.sqrt(1.0 + EPS) * bng_ref[...] + bnb_ref[...])
    h_ref[...] = hid_ref[...] + ls_ref[0, 0] * a


def _tc_post(num, den, hid, cb, bng, bnb, ls):
    grid = (NPAD // BR,)
    blk = lambda r, c: pl.BlockSpec((r, c), lambda i: (0, 0))
    row = lambda c: pl.BlockSpec((BR, c), lambda i: (i, 0))
    return pl.pallas_call(
        _post_body,
        grid=grid,
        in_specs=[row(128), row(16), row(128), blk(1, 128), blk(1, 128),
                  blk(1, 128), blk(1, 1)],
        out_specs=row(128),
        out_shape=jax.ShapeDtypeStruct((NPAD, 128), jnp.float32),
    )(num, den, hid, cb, bng, bnb, ls)


def _final_body(psum_ref, pmax_ref, w1_ref, b1_ref, lng_ref, lnb_ref,
                w2_ref, b2_ref, out_ref):
    ps = psum_ref[...]
    summ = jnp.sum(ps[:, :, :128], axis=0)
    cntv = jnp.sum(ps[:, :, 128:144], axis=0)
    cnt = cntv[:, 0:1]
    pm = pmax_ref[...]
    maxv = jnp.max(pm, axis=0)
    maxv = jnp.where(jnp.isfinite(maxv), maxv, 0.0)
    meanp = summ / jnp.maximum(cnt, 1.0)
    pooled = jnp.concatenate([meanp, maxv, summ], axis=1)
    z = _dot(pooled, w1_ref[...]) + b1_ref[...]
    z = _gelu(_ln(z, lng_ref[...], lnb_ref[...]))
    out_ref[...] = _dot(z, w2_ref[...]) + b2_ref[...]


def _tc_final(psum, pmax, w1, b1, lng, lnb, w2p, b2p):
    full = lambda s: pl.BlockSpec(s, lambda: tuple(0 for _ in s))
    return pl.pallas_call(
        _final_body,
        in_specs=[full((NW, NG, 144)), full((NW, NG, 128)), full((384, 128)),
                  full((1, 128)), full((1, 128)), full((1, 128)),
                  full((128, 128)), full((1, 128))],
        out_specs=full((NG, 128)),
        out_shape=jax.ShapeDtypeStruct((NG, 128), jnp.float32),
    )(psum, pmax, w1, b1, lng, lnb, w2p, b2p)


def kernel(x, params, edge_index, batch):
    p = params
    r1 = lambda v: v.reshape(1, -1)
    xp = jnp.pad(x, ((0, NPAD - NN), (0, 0)))
    batch_p = jnp.pad(batch, (0, NPAD - NN))
    esrc, edst, ecnt = _partition(edge_index)

    identity, hs, hd = _tc_pre(xp, p['proj_W'], r1(p['proj_b']),
                               p['c1_Ws'], p['c1_Wd'])
    num1, den1 = _edge(hs, hd, esrc, edst, ecnt, p['c1_att'].reshape(-1))
    h, hs2, hd2 = _tc_mid(num1, den1.reshape(NPAD, 16), identity, r1(p['c1_b']), r1(p['bn1_g']),
                          r1(p['bn1_b']), p['ls1'].reshape(1, 1),
                          p['ff1_W1'], r1(p['ff1_b1']), r1(p['ff1_lng']),
                          r1(p['ff1_lnb']), p['ff1_W2'], r1(p['ff1_b2']),
                          p['c2_Ws'], p['c2_Wd'])
    num2, den2 = _edge(hs2, hd2, esrc, edst, ecnt, p['c2_att'].reshape(-1))
    h2, hs3, hd3 = _tc_mid(num2, den2.reshape(NPAD, 16), h, r1(p['c2_b']), r1(p['bn2_g']),
                           r1(p['bn2_b']), p['ls2'].reshape(1, 1),
                           p['ff2_W1'], r1(p['ff2_b1']), r1(p['ff2_lng']),
                           r1(p['ff2_lnb']), p['ff2_W2'], r1(p['ff2_b2']),
                           p['c3_Ws'], p['c3_Wd'])
    num3, den3 = _edge(hs3, hd3, esrc, edst, ecnt, p['c3_att'].reshape(-1))
    h3 = _tc_post(num3, den3.reshape(NPAD, 16), h2, r1(p['c3_b']), r1(p['bn3_g']),
                  r1(p['bn3_b']), p['ls3'].reshape(1, 1))

    psum, pmax = _pool(h3, batch_p)

    w2p = jnp.pad(p['out_W2'], ((0, 0), (0, 128 - NCLS)))
    b2p = jnp.pad(p['out_b2'], (0, 128 - NCLS))
    out = _tc_final(psum, pmax, p['out_W1'], r1(p['out_b1']),
                    r1(p['out_lng']), r1(p['out_lnb']), w2p, r1(b2p))
    return out[:, :NCLS]


# speedup vs baseline: 1.0187x; 1.0187x over previous
import functools

import jax
import jax.numpy as jnp
from jax import lax
from jax.experimental import pallas as pl
from jax.experimental.pallas import tpu as pltpu
from jax.experimental.pallas import tpu_sc as plsc

NN = 10000
NE = 320000
HID = 16
HEADS = 8
FFD = HID * HEADS
NG = 64
NCLS = 10
EPS = 1e-5

NC, NS = 2, 16
NW = NC * NS
NPAD = 10240
RT = NPAD // NW
CAP = 16384
CHUNK = 12800
EC = 128

_mesh = plsc.VectorSubcoreMesh(core_axis_name="c", subcore_axis_name="s")
_sc_params = pltpu.CompilerParams(needs_layout_passes=False)


def _wid():
    return lax.axis_index("s") * NC + lax.axis_index("c")


def _partition_body(ei_hbm, esrc_hbm, edst_hbm, ecnt_hbm,
                    ei_v, out_src, out_dst, cnt_v):
    wid = _wid()
    lo = wid * RT
    lane = lax.iota(jnp.int32, 16)

    def chunk_body(ci, cnt):
        pltpu.sync_copy(ei_hbm.at[:, pl.ds(ci * CHUNK, CHUNK)], ei_v)

        def grp(gi, cnt):
            for u in range(4):
                d16 = ei_v[1, pl.ds(gi * 64 + u * 16, 16)]
                s16 = ei_v[0, pl.ds(gi * 64 + u * 16, 16)]
                m = (d16 >= lo) & (d16 < lo + RT)
                pos = plsc.cumsum(m.astype(jnp.int32))
                idx = jnp.where(m, cnt + pos - 1, CAP + lane)
                plsc.store_scatter(out_src, [idx], s16)
                plsc.store_scatter(out_dst, [idx], d16 - lo)
                cnt = cnt + pos[15]
            return cnt

        return lax.fori_loop(0, CHUNK // 64, grp, cnt)

    cnt = lax.fori_loop(0, NE // CHUNK, chunk_body, jnp.int32(0))

    z16 = jnp.zeros((16,), jnp.int32)
    d16 = jnp.full((16,), RT, jnp.int32)

    def ztail(i, _):
        out_src[pl.ds(cnt + i * 16, 16)] = z16
        out_dst[pl.ds(cnt + i * 16, 16)] = d16
        return 0

    lax.fori_loop(0, EC // 16, ztail, 0)

    cnt_v[...] = jnp.full((16,), cnt, jnp.int32)
    pltpu.sync_copy(cnt_v, ecnt_hbm.at[wid])
    pltpu.sync_copy(out_src.at[pl.ds(0, CAP)], esrc_hbm.at[wid])
    pltpu.sync_copy(out_dst.at[pl.ds(0, CAP)], edst_hbm.at[wid])


_partition = functools.partial(
    pl.kernel,
    mesh=_mesh,
    compiler_params=_sc_params,
    out_type=[
        jax.ShapeDtypeStruct((NW, CAP), jnp.int32),
        jax.ShapeDtypeStruct((NW, CAP), jnp.int32),
        jax.ShapeDtypeStruct((NW, 16), jnp.int32),
    ],
    scratch_types=[
        pltpu.VMEM((2, CHUNK), jnp.int32),
        pltpu.VMEM((CAP + 16,), jnp.int32),
        pltpu.VMEM((CAP + 16,), jnp.int32),
        pltpu.VMEM((16,), jnp.int32),
    ],
)(_partition_body)


def _edge_body(hs_hbm, hd_hbm, esrc_hbm, edst_hbm, ecnt_hbm, att_hbm,
               num_hbm, den_hbm, hd_loc, acc_num, acc_den, rows0, rows1,
               srcv0, srcv1, dstv0, dstv1, attv, cntv, sem0, sem1):
    wid = _wid()
    base = wid * RT
    pltpu.sync_copy(hd_hbm.at[pl.ds(base, RT)], hd_loc.at[pl.ds(0, RT)])
    pltpu.sync_copy(att_hbm, attv)
    pltpu.sync_copy(ecnt_hbm.at[wid], cntv)
    cnt = cntv[...][0]

    zf = jnp.zeros((16,), jnp.float32)

    def zrow(r, _):
        for g in range(8):
            acc_num[r, pl.ds(g * 16, 16)] = zf
        acc_den[pl.ds(r * 16, 16)] = zf
        return 0

    lax.fori_loop(0, RT, zrow, 0)
    for g in range(8):
        hd_loc[RT, pl.ds(g * 16, 16)] = zf

    atts = [attv[pl.ds(h * 16, 16)] for h in range(HEADS)]
    lane = lax.iota(jnp.int32, 16)
    i15 = jnp.full((16,), 15, jnp.int32)
    ih = [jnp.full((16,), h, jnp.int32) for h in range(HEADS)]

    nch = (cnt + EC - 1) // EC

    def fetch(ci, srcv, dstv, rows, sem):
        off = ci * EC
        pltpu.sync_copy(esrc_hbm.at[wid, pl.ds(off, EC)], srcv)
        pltpu.sync_copy(edst_hbm.at[wid, pl.ds(off, EC)], dstv.at[pl.ds(0, EC)])
        pltpu.async_copy(hs_hbm.at[srcv], rows, sem)

    def compute(rows, dstv):
        def grp(g, _):
            dl16 = dstv[pl.ds(g * 4, 16)]
            for j in range(4):
                e = g * 4 + j
                dl = dl16[j]
                lv = zf
                for h in range(HEADS):
                    a = rows[e, pl.ds(h * 16, 16)]
                    b = hd_loc[dl, pl.ds(h * 16, 16)]
                    t = a + b
                    t = jnp.maximum(t, 0.2 * t)
                    c = plsc.cumsum(t * atts[h])
                    tot = c[i15]
                    lv = jnp.where(lane == h, tot, lv)
                wv = jnp.exp(lv)
                plsc.addupdate(acc_den.at[pl.ds(dl * 16, 16)], wv)
                for h in range(HEADS):
                    a = rows[e, pl.ds(h * 16, 16)]
                    wh = wv[ih[h]]
                    plsc.addupdate(acc_num.at[dl, pl.ds(h * 16, 16)], a * wh)
            return 0

        lax.fori_loop(0, EC // 4, grp, 0)

    @pl.when(nch > 0)
    def _():
        fetch(0, srcv0, dstv0, rows0, sem0)

    def pair(i, _):
        c0 = 2 * i

        @pl.when(c0 + 1 < nch)
        def _():
            fetch(c0 + 1, srcv1, dstv1, rows1, sem1)

        pltpu.make_async_copy(hs_hbm.at[srcv0], rows0, sem0).wait()
        compute(rows0, dstv0)

        @pl.when(c0 + 2 < nch)
        def _():
            fetch(c0 + 2, srcv0, dstv0, rows0, sem0)

        @pl.when(c0 + 1 < nch)
        def _():
            pltpu.make_async_copy(hs_hbm.at[srcv1], rows1, sem1).wait()
            compute(rows1, dstv1)

        return 0

    lax.fori_loop(0, (nch + 1) // 2, pair, 0)
    pltpu.sync_copy(acc_num.at[pl.ds(0, RT)], num_hbm.at[pl.ds(base, RT)])
    pltpu.sync_copy(acc_den.at[pl.ds(0, RT * 16)], den_hbm.at[wid])


_edge = functools.partial(
    pl.kernel,
    mesh=_mesh,
    compiler_params=_sc_params,
    out_type=[
        jax.ShapeDtypeStruct((NPAD, 128), jnp.float32),
        jax.ShapeDtypeStruct((NW, RT * 16), jnp.float32),
    ],
    scratch_types=[
        pltpu.VMEM((RT + 8, 128), jnp.float32),
        pltpu.VMEM((RT + 8, 128), jnp.float32),
        pltpu.VMEM(((RT + 8) * 16,), jnp.float32),
        pltpu.VMEM((EC, 128), jnp.float32),
        pltpu.VMEM((EC, 128), jnp.float32),
        pltpu.VMEM((EC,), jnp.int32),
        pltpu.VMEM((EC,), jnp.int32),
        pltpu.VMEM((EC + 16,), jnp.int32),
        pltpu.VMEM((EC + 16,), jnp.int32),
        pltpu.VMEM((128,), jnp.float32),
        pltpu.VMEM((16,), jnp.int32),
        pltpu.SemaphoreType.DMA,
        pltpu.SemaphoreType.DMA,
    ],
)(_edge_body)


def _pool_body(h_hbm, batch_hbm, psum_hbm, pmax_hbm, hl, bl, asum, amax):
    wid = _wid()
    base = wid * RT
    pltpu.sync_copy(h_hbm.at[pl.ds(base, RT)], hl)
    pltpu.sync_copy(batch_hbm.at[pl.ds(base, RT)], bl.at[pl.ds(0, RT)])

    zf = jnp.zeros((16,), jnp.float32)
    ninf = jnp.full((16,), -jnp.inf, jnp.float32)

    def zrow(g, _):
        for j in range(9):
            asum[g, pl.ds(j * 16, 16)] = zf
        for j in range(8):
            amax[g, pl.ds(j * 16, 16)] = ninf
        return 0

    lax.fori_loop(0, NG, zrow, 0)

    lane = lax.iota(jnp.int32, 16)
    one0 = jnp.where(lane == 0, 1.0, 0.0).astype(jnp.float32)
    nrows = jnp.minimum(RT, NN - base)

    def row(r, _):
        g = bl[pl.ds(r, 16)][0]
        for j in range(8):
            v = hl[r, pl.ds(j * 16, 16)]
            plsc.addupdate(asum.at[g, pl.ds(j * 16, 16)], v)
            amax[g, pl.ds(j * 16, 16)] = jnp.maximum(amax[g, pl.ds(j * 16, 16)], v)
        plsc.addupdate(asum.at[g, pl.ds(128, 16)], one0)
        return 0

    lax.fori_loop(0, nrows, row, 0)

    pltpu.sync_copy(asum, psum_hbm.at[wid])
    pltpu.sync_copy(amax, pmax_hbm.at[wid])


_pool = functools.partial(
    pl.kernel,
    mesh=_mesh,
    compiler_params=_sc_params,
    out_type=[
        jax.ShapeDtypeStruct((NW, NG, 144), jnp.float32),
        jax.ShapeDtypeStruct((NW, NG, 128), jnp.float32),
    ],
    scratch_types=[
        pltpu.VMEM((RT, 128), jnp.float32),
        pltpu.VMEM((RT + 16,), jnp.int32),
        pltpu.VMEM((NG, 144), jnp.float32),
        pltpu.VMEM((NG, 128), jnp.float32),
    ],
)(_pool_body)


_PREC = lax.Precision.HIGHEST


def _dot(a, b):
    return jnp.dot(a, b, precision=_PREC, preferred_element_type=jnp.float32)


def _ln(z, g, b):
    mu = jnp.mean(z, axis=-1, keepdims=True)
    var = jnp.mean((z - mu) ** 2, axis=-1, keepdims=True)
    return (z - mu) / jnp.sqrt(var + EPS) * g + b


def _gelu(z):
    return 0.5 * z * (1.0 + lax.erf(z * 0.7071067811865476))


BR = 1024


def _pre_body(x_ref, pw_ref, pb_ref, ws_ref, wd_ref, id_ref, hs_ref, hd_ref):
    x = x_ref[...]
    id_ref[...] = _dot(x, pw_ref[...]) + pb_ref[...]
    hs_ref[...] = _dot(x, ws_ref[...])
    hd_ref[...] = _dot(x, wd_ref[...])


def _tc_pre(xp, pw, pb, ws, wd):
    grid = (NPAD // BR,)
    blk = lambda r, c: pl.BlockSpec((r, c), lambda i: (0, 0))
    row = lambda c: pl.BlockSpec((BR, c), lambda i: (i, 0))
    return pl.pallas_call(
        _pre_body,
        grid=grid,
        in_specs=[row(128), blk(128, 128), blk(1, 128), blk(128, 128), blk(128, 128)],
        out_specs=[row(128), row(128), row(128)],
        out_shape=[jax.ShapeDtypeStruct((NPAD, 128), jnp.float32)] * 3,
    )(xp, pw, pb, ws, wd)


def _mid_body(has_next, num_ref, den_ref, hid_ref, cb_ref, bng_ref, bnb_ref, ls_ref,
              w1_ref, b1_ref, lng_ref, lnb_ref, w2_ref, b2_ref, *rest):
    if has_next:
        wsn_ref, wdn_ref, h_ref, hs_ref, hd_ref = rest
    else:
        (h_ref,) = rest
    num = num_ref[...]
    den16 = den_ref[...]
    sel = (lax.broadcasted_iota(jnp.int32, (16, 128), 1) // 16
           == lax.broadcasted_iota(jnp.int32, (16, 128), 0)).astype(jnp.float32)
    den = _dot(den16, sel)
    gat = num / (den + 1e-16) + cb_ref[...]
    a = _gelu(gat / jnp.sqrt(1.0 + EPS) * bng_ref[...] + bnb_ref[...])
    ls = ls_ref[0, 0]
    h = hid_ref[...] + ls * a
    z = _dot(h, w1_ref[...]) + b1_ref[...]
    z = _ln(z, lng_ref[...], lnb_ref[...])
    z = _gelu(z)
    ff = _dot(z, w2_ref[...]) + b2_ref[...]
    h2 = h + ls * ff
    h_ref[...] = h2
    if has_next:
        hs_ref[...] = _dot(h2, wsn_ref[...])
        hd_ref[...] = _dot(h2, wdn_ref[...])


def _tc_mid(num, den, hid, cb, bng, bnb, ls, w1, b1, lng, lnb, w2, b2, wsn=None, wdn=None):
    has_next = wsn is not None
    grid = (NPAD // BR,)
    blk = lambda r, c: pl.BlockSpec((r, c), lambda i: (0, 0))
    row = lambda c: pl.BlockSpec((BR, c), lambda i: (i, 0))
    in_specs = [row(128), row(16), row(128), blk(1, 128), blk(1, 128), blk(1, 128),
                blk(1, 1), blk(128, 256), blk(1, 256), blk(1, 256), blk(1, 256),
                blk(256, 128), blk(1, 128)]
    args = [num, den, hid, cb, bng, bnb, ls, w1, b1, lng, lnb, w2, b2]
    if has_next:
        in_specs += [blk(128, 128), blk(128, 128)]
        args += [wsn, wdn]
        out_specs = [row(128)] * 3
        out_shape = [jax.ShapeDtypeStruct((NPAD, 128), jnp.float32)] * 3
    else:
        out_specs = [row(128)]
        out_shape = [jax.ShapeDtypeStruct((NPAD, 128), jnp.float32)]
    return pl.pallas_call(
        functools.partial(_mid_body, has_next),
        grid=grid,
        in_specs=in_specs,
        out_specs=out_specs,
        out_shape=out_shape,
    )(*args)


def _post_body(num_ref, den_ref, hid_ref, cb_ref, bng_ref, bnb_ref, ls_ref, h_ref):
    num = num_ref[...]
    den16 = den_ref[...]
    sel = (lax.broadcasted_iota(jnp.int32, (16, 128), 1) // 16
           == lax.broadcasted_iota(jnp.int32, (16, 128), 0)).astype(jnp.float32)
    den = _dot(den16, sel)
    gat = num / (den + 1e-16) + cb_ref[...]
    a = _gelu(gat / jnp.sqrt(1.0 + EPS) * bng_ref[...] + bnb_ref[...])
    h_ref[...] = hid_ref[...] + ls_ref[0, 0] * a


def _tc_post(num, den, hid, cb, bng, bnb, ls):
    grid = (NPAD // BR,)
    blk = lambda r, c: pl.BlockSpec((r, c), lambda i: (0, 0))
    row = lambda c: pl.BlockSpec((BR, c), lambda i: (i, 0))
    return pl.pallas_call(
        _post_body,
        grid=grid,
        in_specs=[row(128), row(16), row(128), blk(1, 128), blk(1, 128),
                  blk(1, 128), blk(1, 1)],
        out_specs=row(128),
        out_shape=jax.ShapeDtypeStruct((NPAD, 128), jnp.float32),
    )(num, den, hid, cb, bng, bnb, ls)


def _final_body(psum_ref, pmax_ref, w1_ref, b1_ref, lng_ref, lnb_ref,
                w2_ref, b2_ref, out_ref):
    ps = psum_ref[...]
    summ = jnp.sum(ps[:, :, :128], axis=0)
    cntv = jnp.sum(ps[:, :, 128:144], axis=0)
    cnt = cntv[:, 0:1]
    pm = pmax_ref[...]
    maxv = jnp.max(pm, axis=0)
    maxv = jnp.where(jnp.isfinite(maxv), maxv, 0.0)
    meanp = summ / jnp.maximum(cnt, 1.0)
    pooled = jnp.concatenate([meanp, maxv, summ], axis=1)
    z = _dot(pooled, w1_ref[...]) + b1_ref[...]
    z = _gelu(_ln(z, lng_ref[...], lnb_ref[...]))
    out_ref[...] = _dot(z, w2_ref[...]) + b2_ref[...]


def _tc_final(psum, pmax, w1, b1, lng, lnb, w2p, b2p):
    full = lambda s: pl.BlockSpec(s, lambda: tuple(0 for _ in s))
    return pl.pallas_call(
        _final_body,
        in_specs=[full((NW, NG, 144)), full((NW, NG, 128)), full((384, 128)),
                  full((1, 128)), full((1, 128)), full((1, 128)),
                  full((128, 128)), full((1, 128))],
        out_specs=full((NG, 128)),
        out_shape=jax.ShapeDtypeStruct((NG, 128), jnp.float32),
    )(psum, pmax, w1, b1, lng, lnb, w2p, b2p)


def kernel(x, params, edge_index, batch):
    p = params
    r1 = lambda v: v.reshape(1, -1)
    xp = jnp.pad(x, ((0, NPAD - NN), (0, 0)))
    batch_p = jnp.pad(batch, (0, NPAD - NN))
    esrc, edst, ecnt = _partition(edge_index)

    identity, hs, hd = _tc_pre(xp, p['proj_W'], r1(p['proj_b']),
                               p['c1_Ws'], p['c1_Wd'])
    num1, den1 = _edge(hs, hd, esrc, edst, ecnt, p['c1_att'].reshape(-1))
    h, hs2, hd2 = _tc_mid(num1, den1.reshape(NPAD, 16), identity, r1(p['c1_b']), r1(p['bn1_g']),
                          r1(p['bn1_b']), p['ls1'].reshape(1, 1),
                          p['ff1_W1'], r1(p['ff1_b1']), r1(p['ff1_lng']),
                          r1(p['ff1_lnb']), p['ff1_W2'], r1(p['ff1_b2']),
                          p['c2_Ws'], p['c2_Wd'])
    num2, den2 = _edge(hs2, hd2, esrc, edst, ecnt, p['c2_att'].reshape(-1))
    h2, hs3, hd3 = _tc_mid(num2, den2.reshape(NPAD, 16), h, r1(p['c2_b']), r1(p['bn2_g']),
                           r1(p['bn2_b']), p['ls2'].reshape(1, 1),
                           p['ff2_W1'], r1(p['ff2_b1']), r1(p['ff2_lng']),
                           r1(p['ff2_lnb']), p['ff2_W2'], r1(p['ff2_b2']),
                           p['c3_Ws'], p['c3_Wd'])
    num3, den3 = _edge(hs3, hd3, esrc, edst, ecnt, p['c3_att'].reshape(-1))
    h3 = _tc_post(num3, den3.reshape(NPAD, 16), h2, r1(p['c3_b']), r1(p['bn3_g']),
                  r1(p['bn3_b']), p['ls3'].reshape(1, 1))

    psum, pmax = _pool(h3, batch_p)

    w2p = jnp.pad(p['out_W2'], ((0, 0), (0, 128 - NCLS)))
    b2p = jnp.pad(p['out_b2'], (0, 128 - NCLS))
    out = _tc_final(psum, pmax, p['out_W1'], r1(p['out_b1']),
                    r1(p['out_lng']), r1(p['out_lnb']), w2p, r1(b2p))
    return out[:, :NCLS]


# speedup vs baseline: 1.0300x; 1.0111x over previous
import functools

import jax
import jax.numpy as jnp
from jax import lax
from jax.experimental import pallas as pl
from jax.experimental.pallas import tpu as pltpu
from jax.experimental.pallas import tpu_sc as plsc

NN = 10000
NE = 320000
HID = 16
HEADS = 8
FFD = HID * HEADS
NG = 64
NCLS = 10
EPS = 1e-5

NC, NS = 2, 16
NW = NC * NS
NPAD = 10240
RT = NPAD // NW
CAP = 16384
CHUNK = 12800
EC = 128

_mesh = plsc.VectorSubcoreMesh(core_axis_name="c", subcore_axis_name="s")
_sc_params = pltpu.CompilerParams(needs_layout_passes=False)


def _wid():
    return lax.axis_index("s") * NC + lax.axis_index("c")


def _partition_body(ei_hbm, esrc_hbm, edst_hbm, ecnt_hbm,
                    ei_v, out_src, out_dst, cnt_v):
    wid = _wid()
    lo = wid * RT
    lane = lax.iota(jnp.int32, 16)

    def chunk_body(ci, cnt):
        pltpu.sync_copy(ei_hbm.at[:, pl.ds(ci * CHUNK, CHUNK)], ei_v)

        def grp(gi, cnt):
            for u in range(4):
                d16 = ei_v[1, pl.ds(gi * 64 + u * 16, 16)]
                s16 = ei_v[0, pl.ds(gi * 64 + u * 16, 16)]
                m = (d16 >= lo) & (d16 < lo + RT)
                pos = plsc.cumsum(m.astype(jnp.int32))
                idx = jnp.where(m, cnt + pos - 1, CAP + lane)
                plsc.store_scatter(out_src, [idx], s16)
                plsc.store_scatter(out_dst, [idx], d16 - lo)
                cnt = cnt + pos[15]
            return cnt

        return lax.fori_loop(0, CHUNK // 64, grp, cnt)

    cnt = lax.fori_loop(0, NE // CHUNK, chunk_body, jnp.int32(0))

    z16 = jnp.zeros((16,), jnp.int32)
    d16 = jnp.full((16,), RT, jnp.int32)

    def ztail(i, _):
        out_src[pl.ds(cnt + i * 16, 16)] = z16
        out_dst[pl.ds(cnt + i * 16, 16)] = d16
        return 0

    lax.fori_loop(0, EC // 16, ztail, 0)

    cnt_v[...] = jnp.full((16,), cnt, jnp.int32)
    pltpu.sync_copy(cnt_v, ecnt_hbm.at[wid])
    pltpu.sync_copy(out_src.at[pl.ds(0, CAP)], esrc_hbm.at[wid])
    pltpu.sync_copy(out_dst.at[pl.ds(0, CAP)], edst_hbm.at[wid])


_partition = functools.partial(
    pl.kernel,
    mesh=_mesh,
    compiler_params=_sc_params,
    out_type=[
        jax.ShapeDtypeStruct((NW, CAP), jnp.int32),
        jax.ShapeDtypeStruct((NW, CAP), jnp.int32),
        jax.ShapeDtypeStruct((NW, 16), jnp.int32),
    ],
    scratch_types=[
        pltpu.VMEM((2, CHUNK), jnp.int32),
        pltpu.VMEM((CAP + 16,), jnp.int32),
        pltpu.VMEM((CAP + 16,), jnp.int32),
        pltpu.VMEM((16,), jnp.int32),
    ],
)(_partition_body)


def _edge_body(hs_hbm, hd_hbm, esrc_hbm, edst_hbm, ecnt_hbm, att_hbm,
               num_hbm, den_hbm, hd_loc, acc_num, acc_den, rows0, rows1,
               srcv0, srcv1, dstv0, dstv1, attv, cntv, sem0, sem1):
    wid = _wid()
    base = wid * RT
    pltpu.sync_copy(hd_hbm.at[pl.ds(base, RT)], hd_loc.at[pl.ds(0, RT)])
    pltpu.sync_copy(att_hbm, attv)
    pltpu.sync_copy(ecnt_hbm.at[wid], cntv)
    cnt = cntv[...][0]

    zf = jnp.zeros((16,), jnp.float32)

    def zrow(r, _):
        for g in range(8):
            acc_num[r, pl.ds(g * 16, 16)] = zf
        acc_den[pl.ds(r * 16, 16)] = zf
        return 0

    lax.fori_loop(0, RT, zrow, 0)
    for g in range(8):
        hd_loc[RT, pl.ds(g * 16, 16)] = zf

    atts = [attv[pl.ds(h * 16, 16)] for h in range(HEADS)]
    lane = lax.iota(jnp.int32, 16)
    i15 = jnp.full((16,), 15, jnp.int32)
    ih = [jnp.full((16,), h, jnp.int32) for h in range(HEADS)]

    nch = (cnt + EC - 1) // EC

    def fetch(ci, srcv, dstv, rows, sem):
        off = ci * EC
        pltpu.sync_copy(esrc_hbm.at[wid, pl.ds(off, EC)], srcv)
        pltpu.sync_copy(edst_hbm.at[wid, pl.ds(off, EC)], dstv.at[pl.ds(0, EC)])
        pltpu.async_copy(hs_hbm.at[srcv], rows, sem)

    def compute(rows, dstv):
        def grp(g, _):
            dl16 = dstv[pl.ds(g * 8, 16)]
            for j in range(8):
                e = g * 8 + j
                dl = dl16[j]
                lv = zf
                for h in range(HEADS):
                    a = rows[e, pl.ds(h * 16, 16)]
                    b = hd_loc[dl, pl.ds(h * 16, 16)]
                    t = a + b
                    t = jnp.maximum(t, 0.2 * t)
                    c = plsc.cumsum(t * atts[h])
                    tot = c[i15]
                    lv = jnp.where(lane == h, tot, lv)
                wv = jnp.exp(lv)
                plsc.addupdate(acc_den.at[pl.ds(dl * 16, 16)], wv)
                for h in range(HEADS):
                    a = rows[e, pl.ds(h * 16, 16)]
                    wh = wv[ih[h]]
                    plsc.addupdate(acc_num.at[dl, pl.ds(h * 16, 16)], a * wh)
            return 0

        lax.fori_loop(0, EC // 8, grp, 0)

    @pl.when(nch > 0)
    def _():
        fetch(0, srcv0, dstv0, rows0, sem0)

    def pair(i, _):
        c0 = 2 * i

        @pl.when(c0 + 1 < nch)
        def _():
            fetch(c0 + 1, srcv1, dstv1, rows1, sem1)

        pltpu.make_async_copy(hs_hbm.at[srcv0], rows0, sem0).wait()
        compute(rows0, dstv0)

        @pl.when(c0 + 2 < nch)
        def _():
            fetch(c0 + 2, srcv0, dstv0, rows0, sem0)

        @pl.when(c0 + 1 < nch)
        def _():
            pltpu.make_async_copy(hs_hbm.at[srcv1], rows1, sem1).wait()
            compute(rows1, dstv1)

        return 0

    lax.fori_loop(0, (nch + 1) // 2, pair, 0)
    pltpu.sync_copy(acc_num.at[pl.ds(0, RT)], num_hbm.at[pl.ds(base, RT)])
    pltpu.sync_copy(acc_den.at[pl.ds(0, RT * 16)], den_hbm.at[wid])


_edge = functools.partial(
    pl.kernel,
    mesh=_mesh,
    compiler_params=_sc_params,
    out_type=[
        jax.ShapeDtypeStruct((NPAD, 128), jnp.float32),
        jax.ShapeDtypeStruct((NW, RT * 16), jnp.float32),
    ],
    scratch_types=[
        pltpu.VMEM((RT + 8, 128), jnp.float32),
        pltpu.VMEM((RT + 8, 128), jnp.float32),
        pltpu.VMEM(((RT + 8) * 16,), jnp.float32),
        pltpu.VMEM((EC, 128), jnp.float32),
        pltpu.VMEM((EC, 128), jnp.float32),
        pltpu.VMEM((EC,), jnp.int32),
        pltpu.VMEM((EC,), jnp.int32),
        pltpu.VMEM((EC + 16,), jnp.int32),
        pltpu.VMEM((EC + 16,), jnp.int32),
        pltpu.VMEM((128,), jnp.float32),
        pltpu.VMEM((16,), jnp.int32),
        pltpu.SemaphoreType.DMA,
        pltpu.SemaphoreType.DMA,
    ],
)(_edge_body)


def _pool_body(h_hbm, batch_hbm, psum_hbm, pmax_hbm, hl, bl, asum, amax):
    wid = _wid()
    base = wid * RT
    pltpu.sync_copy(h_hbm.at[pl.ds(base, RT)], hl)
    pltpu.sync_copy(batch_hbm.at[pl.ds(base, RT)], bl.at[pl.ds(0, RT)])

    zf = jnp.zeros((16,), jnp.float32)
    ninf = jnp.full((16,), -jnp.inf, jnp.float32)

    def zrow(g, _):
        for j in range(9):
            asum[g, pl.ds(j * 16, 16)] = zf
        for j in range(8):
            amax[g, pl.ds(j * 16, 16)] = ninf
        return 0

    lax.fori_loop(0, NG, zrow, 0)

    lane = lax.iota(jnp.int32, 16)
    one0 = jnp.where(lane == 0, 1.0, 0.0).astype(jnp.float32)
    nrows = jnp.minimum(RT, NN - base)

    def row(r, _):
        g = bl[pl.ds(r, 16)][0]
        for j in range(8):
            v = hl[r, pl.ds(j * 16, 16)]
            plsc.addupdate(asum.at[g, pl.ds(j * 16, 16)], v)
            amax[g, pl.ds(j * 16, 16)] = jnp.maximum(amax[g, pl.ds(j * 16, 16)], v)
        plsc.addupdate(asum.at[g, pl.ds(128, 16)], one0)
        return 0

    lax.fori_loop(0, nrows, row, 0)

    pltpu.sync_copy(asum, psum_hbm.at[wid])
    pltpu.sync_copy(amax, pmax_hbm.at[wid])


_pool = functools.partial(
    pl.kernel,
    mesh=_mesh,
    compiler_params=_sc_params,
    out_type=[
        jax.ShapeDtypeStruct((NW, NG, 144), jnp.float32),
        jax.ShapeDtypeStruct((NW, NG, 128), jnp.float32),
    ],
    scratch_types=[
        pltpu.VMEM((RT, 128), jnp.float32),
        pltpu.VMEM((RT + 16,), jnp.int32),
        pltpu.VMEM((NG, 144), jnp.float32),
        pltpu.VMEM((NG, 128), jnp.float32),
    ],
)(_pool_body)


_PREC = lax.Precision.HIGHEST


def _dot(a, b):
    return jnp.dot(a, b, precision=_PREC, preferred_element_type=jnp.float32)


def _ln(z, g, b):
    mu = jnp.mean(z, axis=-1, keepdims=True)
    var = jnp.mean((z - mu) ** 2, axis=-1, keepdims=True)
    return (z - mu) / jnp.sqrt(var + EPS) * g + b


def _gelu(z):
    return 0.5 * z * (1.0 + lax.erf(z * 0.7071067811865476))


BR = 1024


def _pre_body(x_ref, pw_ref, pb_ref, ws_ref, wd_ref, id_ref, hs_ref, hd_ref):
    x = x_ref[...]
    id_ref[...] = _dot(x, pw_ref[...]) + pb_ref[...]
    hs_ref[...] = _dot(x, ws_ref[...])
    hd_ref[...] = _dot(x, wd_ref[...])


def _tc_pre(xp, pw, pb, ws, wd):
    grid = (NPAD // BR,)
    blk = lambda r, c: pl.BlockSpec((r, c), lambda i: (0, 0))
    row = lambda c: pl.BlockSpec((BR, c), lambda i: (i, 0))
    return pl.pallas_call(
        _pre_body,
        grid=grid,
        in_specs=[row(128), blk(128, 128), blk(1, 128), blk(128, 128), blk(128, 128)],
        out_specs=[row(128), row(128), row(128)],
        out_shape=[jax.ShapeDtypeStruct((NPAD, 128), jnp.float32)] * 3,
    )(xp, pw, pb, ws, wd)


def _mid_body(has_next, num_ref, den_ref, hid_ref, cb_ref, bng_ref, bnb_ref, ls_ref,
              w1_ref, b1_ref, lng_ref, lnb_ref, w2_ref, b2_ref, *rest):
    if has_next:
        wsn_ref, wdn_ref, h_ref, hs_ref, hd_ref = rest
    else:
        (h_ref,) = rest
    num = num_ref[...]
    den16 = den_ref[...]
    sel = (lax.broadcasted_iota(jnp.int32, (16, 128), 1) // 16
           == lax.broadcasted_iota(jnp.int32, (16, 128), 0)).astype(jnp.float32)
    den = _dot(den16, sel)
    gat = num / (den + 1e-16) + cb_ref[...]
    a = _gelu(gat / jnp.sqrt(1.0 + EPS) * bng_ref[...] + bnb_ref[...])
    ls = ls_ref[0, 0]
    h = hid_ref[...] + ls * a
    z = _dot(h, w1_ref[...]) + b1_ref[...]
    z = _ln(z, lng_ref[...], lnb_ref[...])
    z = _gelu(z)
    ff = _dot(z, w2_ref[...]) + b2_ref[...]
    h2 = h + ls * ff
    h_ref[...] = h2
    if has_next:
        hs_ref[...] = _dot(h2, wsn_ref[...])
        hd_ref[...] = _dot(h2, wdn_ref[...])


def _tc_mid(num, den, hid, cb, bng, bnb, ls, w1, b1, lng, lnb, w2, b2, wsn=None, wdn=None):
    has_next = wsn is not None
    grid = (NPAD // BR,)
    blk = lambda r, c: pl.BlockSpec((r, c), lambda i: (0, 0))
    row = lambda c: pl.BlockSpec((BR, c), lambda i: (i, 0))
    in_specs = [row(128), row(16), row(128), blk(1, 128), blk(1, 128), blk(1, 128),
                blk(1, 1), blk(128, 256), blk(1, 256), blk(1, 256), blk(1, 256),
                blk(256, 128), blk(1, 128)]
    args = [num, den, hid, cb, bng, bnb, ls, w1, b1, lng, lnb, w2, b2]
    if has_next:
        in_specs += [blk(128, 128), blk(128, 128)]
        args += [wsn, wdn]
        out_specs = [row(128)] * 3
        out_shape = [jax.ShapeDtypeStruct((NPAD, 128), jnp.float32)] * 3
    else:
        out_specs = [row(128)]
        out_shape = [jax.ShapeDtypeStruct((NPAD, 128), jnp.float32)]
    return pl.pallas_call(
        functools.partial(_mid_body, has_next),
        grid=grid,
        in_specs=in_specs,
        out_specs=out_specs,
        out_shape=out_shape,
    )(*args)


def _post_body(num_ref, den_ref, hid_ref, cb_ref, bng_ref, bnb_ref, ls_ref, h_ref):
    num = num_ref[...]
    den16 = den_ref[...]
    sel = (lax.broadcasted_iota(jnp.int32, (16, 128), 1) // 16
           == lax.broadcasted_iota(jnp.int32, (16, 128), 0)).astype(jnp.float32)
    den = _dot(den16, sel)
    gat = num / (den + 1e-16) + cb_ref[...]
    a = _gelu(gat / jnp.sqrt(1.0 + EPS) * bng_ref[...] + bnb_ref[...])
    h_ref[...] = hid_ref[...] + ls_ref[0, 0] * a


def _tc_post(num, den, hid, cb, bng, bnb, ls):
    grid = (NPAD // BR,)
    blk = lambda r, c: pl.BlockSpec((r, c), lambda i: (0, 0))
    row = lambda c: pl.BlockSpec((BR, c), lambda i: (i, 0))
    return pl.pallas_call(
        _post_body,
        grid=grid,
        in_specs=[row(128), row(16), row(128), blk(1, 128), blk(1, 128),
                  blk(1, 128), blk(1, 1)],
        out_specs=row(128),
        out_shape=jax.ShapeDtypeStruct((NPAD, 128), jnp.float32),
    )(num, den, hid, cb, bng, bnb, ls)


def _final_body(psum_ref, pmax_ref, w1_ref, b1_ref, lng_ref, lnb_ref,
                w2_ref, b2_ref, out_ref):
    ps = psum_ref[...]
    summ = jnp.sum(ps[:, :, :128], axis=0)
    cntv = jnp.sum(ps[:, :, 128:144], axis=0)
    cnt = cntv[:, 0:1]
    pm = pmax_ref[...]
    maxv = jnp.max(pm, axis=0)
    maxv = jnp.where(jnp.isfinite(maxv), maxv, 0.0)
    meanp = summ / jnp.maximum(cnt, 1.0)
    pooled = jnp.concatenate([meanp, maxv, summ], axis=1)
    z = _dot(pooled, w1_ref[...]) + b1_ref[...]
    z = _gelu(_ln(z, lng_ref[...], lnb_ref[...]))
    out_ref[...] = _dot(z, w2_ref[...]) + b2_ref[...]


def _tc_final(psum, pmax, w1, b1, lng, lnb, w2p, b2p):
    full = lambda s: pl.BlockSpec(s, lambda: tuple(0 for _ in s))
    return pl.pallas_call(
        _final_body,
        in_specs=[full((NW, NG, 144)), full((NW, NG, 128)), full((384, 128)),
                  full((1, 128)), full((1, 128)), full((1, 128)),
                  full((128, 128)), full((1, 128))],
        out_specs=full((NG, 128)),
        out_shape=jax.ShapeDtypeStruct((NG, 128), jnp.float32),
    )(psum, pmax, w1, b1, lng, lnb, w2p, b2p)


def kernel(x, params, edge_index, batch):
    p = params
    r1 = lambda v: v.reshape(1, -1)
    xp = jnp.pad(x, ((0, NPAD - NN), (0, 0)))
    batch_p = jnp.pad(batch, (0, NPAD - NN))
    esrc, edst, ecnt = _partition(edge_index)

    identity, hs, hd = _tc_pre(xp, p['proj_W'], r1(p['proj_b']),
                               p['c1_Ws'], p['c1_Wd'])
    num1, den1 = _edge(hs, hd, esrc, edst, ecnt, p['c1_att'].reshape(-1))
    h, hs2, hd2 = _tc_mid(num1, den1.reshape(NPAD, 16), identity, r1(p['c1_b']), r1(p['bn1_g']),
                          r1(p['bn1_b']), p['ls1'].reshape(1, 1),
                          p['ff1_W1'], r1(p['ff1_b1']), r1(p['ff1_lng']),
                          r1(p['ff1_lnb']), p['ff1_W2'], r1(p['ff1_b2']),
                          p['c2_Ws'], p['c2_Wd'])
    num2, den2 = _edge(hs2, hd2, esrc, edst, ecnt, p['c2_att'].reshape(-1))
    h2, hs3, hd3 = _tc_mid(num2, den2.reshape(NPAD, 16), h, r1(p['c2_b']), r1(p['bn2_g']),
                           r1(p['bn2_b']), p['ls2'].reshape(1, 1),
                           p['ff2_W1'], r1(p['ff2_b1']), r1(p['ff2_lng']),
                           r1(p['ff2_lnb']), p['ff2_W2'], r1(p['ff2_b2']),
                           p['c3_Ws'], p['c3_Wd'])
    num3, den3 = _edge(hs3, hd3, esrc, edst, ecnt, p['c3_att'].reshape(-1))
    h3 = _tc_post(num3, den3.reshape(NPAD, 16), h2, r1(p['c3_b']), r1(p['bn3_g']),
                  r1(p['bn3_b']), p['ls3'].reshape(1, 1))

    psum, pmax = _pool(h3, batch_p)

    w2p = jnp.pad(p['out_W2'], ((0, 0), (0, 128 - NCLS)))
    b2p = jnp.pad(p['out_b2'], (0, 128 - NCLS))
    out = _tc_final(psum, pmax, p['out_W1'], r1(p['out_b1']),
                    r1(p['out_lng']), r1(p['out_lnb']), w2p, r1(b2p))
    return out[:, :NCLS]
